# baseline (speedup 1.0000x reference)
import jax
import jax.numpy as jnp
from jax import lax
from jax.experimental import pallas as pl
from jax.experimental.pallas import tpu as pltpu
from jax.experimental.pallas import tpu_sc as plsc

BATCH = 16384
DIM = 32
NUM_CORES = 2
NUM_SUBCORES = 16
NUM_WORKERS = NUM_CORES * NUM_SUBCORES
B_PER_W = BATCH // NUM_WORKERS
LANES = 16
N_WINDOWS = 2
ROWS_W = B_PER_W // N_WINDOWS


def _mf_body(u_hbm, i_hbm, uw_hbm, iw_hbm, dummy_hbm, out_hbm,
             u_idx, i_idx, ue_rows, ie_rows, out_v, sem):
    wid = lax.axis_index("s") * NUM_CORES + lax.axis_index("c")

    pltpu.sync_copy(u_hbm.at[pl.ds(wid * B_PER_W, B_PER_W)], u_idx)
    pltpu.sync_copy(i_hbm.at[pl.ds(wid * B_PER_W, B_PER_W)], i_idx)

    lane_iota = lax.iota(jnp.int32, LANES)

    for w in range(N_WINDOWS):
        @pl.loop(0, ROWS_W // LANES)
        def _fire(b):
            base = w * ROWS_W + b * LANES
            uv = u_idx[pl.ds(base, LANES)]
            iv = i_idx[pl.ds(base, LANES)]
            for l in range(LANES):
                dst = b * LANES + l
                pltpu.async_copy(
                    uw_hbm.at[pl.ds(uv[l], 1)],
                    ue_rows.at[pl.ds(dst, 1)], sem)
                pltpu.async_copy(
                    iw_hbm.at[pl.ds(iv[l], 1)],
                    ie_rows.at[pl.ds(dst, 1)], sem)

        pltpu.make_async_copy(dummy_hbm, ue_rows, sem).wait()
        pltpu.make_async_copy(dummy_hbm, ie_rows, sem).wait()

        @pl.loop(0, ROWS_W // LANES)
        def _group(g):
            vrow = g * LANES + lane_iota
            acc = jnp.zeros((LANES,), jnp.float32)
            for k in range(DIM):
                vcol = jnp.full((LANES,), k, jnp.int32)
                gu = plsc.load_gather(ue_rows, [vrow, vcol])
                gi = plsc.load_gather(ie_rows, [vrow, vcol])
                acc = acc + gu * gi
            out_v[pl.ds(w * ROWS_W + g * LANES, LANES)] = acc

    pltpu.sync_copy(out_v, out_hbm.at[pl.ds(wid * B_PER_W, B_PER_W)])


def kernel(u, i, user_weight, item_weight):
    u2 = u.astype(jnp.int32)
    i2 = i.astype(jnp.int32)
    dummy = jnp.zeros((ROWS_W, DIM), jnp.float32)
    mesh = plsc.VectorSubcoreMesh(
        core_axis_name="c", subcore_axis_name="s",
        num_cores=NUM_CORES, num_subcores=NUM_SUBCORES)
    run = pl.kernel(
        _mf_body,
        out_type=jax.ShapeDtypeStruct((BATCH,), jnp.float32),
        mesh=mesh,
        compiler_params=pltpu.CompilerParams(needs_layout_passes=False,
                                             use_tc_tiling_on_sc=True),
        scratch_types=[
            pltpu.VMEM((B_PER_W,), jnp.int32),
            pltpu.VMEM((B_PER_W,), jnp.int32),
            pltpu.VMEM((ROWS_W, DIM), jnp.float32),
            pltpu.VMEM((ROWS_W, DIM), jnp.float32),
            pltpu.VMEM((B_PER_W,), jnp.float32),
            pltpu.SemaphoreType.DMA,
        ],
    )
    return run(u2, i2, user_weight, item_weight, dummy)
